# Initial kernel scaffold; baseline (speedup 1.0000x reference)
import jax
import jax.numpy as jnp
from jax.experimental import pallas as pl


def kernel(x, pos, batch, params):
    raise NotImplementedError("write your pallas kernel here")


# speedup vs baseline: 8.6123x; 8.6123x over previous
import functools

import jax
import jax.numpy as jnp
import numpy as np
from jax.experimental import pallas as pl
from jax.experimental.pallas import tpu as pltpu
from jax.experimental.pallas import tpu_sc as plsc

NB = 4
N0 = 4096
IN_CH = 6
M1, M1P = 819, 1024
M2, M2P = 204, 256
KNB = 64
PADV = 1e9
NEGINF = float('-inf')
POSINF = float('inf')


def _fps_kern(pos_ref, out_ref, *, n, m, mp):
    c8 = pos_ref.shape[3]
    px = pos_ref[0, 0]
    py = pos_ref[0, 1]
    pz = pos_ref[0, 2]
    li = (jax.lax.broadcasted_iota(jnp.int32, (8, c8), 0) * c8
          + jax.lax.broadcasted_iota(jnp.int32, (8, c8), 1))
    mind0 = jnp.where(li < n, POSINF, NEGINF)
    acc0 = jnp.zeros((1, mp), jnp.int32)
    ji = jax.lax.broadcasted_iota(jnp.int32, (1, mp), 1)
    bigi = jnp.int32(2 ** 30)

    def body(i, st):
        mind, acc = st
        mx = jnp.max(mind)
        far = jnp.min(jnp.where(mind == mx, li, bigi))
        sel = li == far
        qx = jnp.sum(jnp.where(sel, px, 0.0))
        qy = jnp.sum(jnp.where(sel, py, 0.0))
        qz = jnp.sum(jnp.where(sel, pz, 0.0))
        dx = px - qx
        dy = py - qy
        dz = pz - qz
        d = (dx * dx + dy * dy) + dz * dz
        mind = jnp.minimum(mind, d)
        acc = jnp.where(ji == i, far, acc)
        return mind, acc

    _, acc = jax.lax.fori_loop(0, m, body, (mind0, acc0))
    out_ref[0] = acc


def _fps(pos4, n, m, mp):
    c8 = pos4.shape[3]
    out = pl.pallas_call(
        functools.partial(_fps_kern, n=n, m=m, mp=mp),
        grid=(NB,),
        in_specs=[pl.BlockSpec((1, 3, 8, c8), lambda b: (b, 0, 0, 0))],
        out_specs=pl.BlockSpec((1, 1, mp), lambda b: (b, 0, 0)),
        out_shape=jax.ShapeDtypeStruct((NB, 1, mp), jnp.int32),
    )(pos4)
    return out[:, 0, :m]


def _nbr_kern(ps_ref, pd_ref, nbr_ref, vm_ref, *, r2, kk):
    ns = ps_ref.shape[2]
    bd = pd_ref.shape[1]
    sx = ps_ref[0, 0:1, :]
    sy = ps_ref[0, 1:2, :]
    sz = ps_ref[0, 2:3, :]
    pd = pd_ref[0]
    ddx = pd[:, 0:1] - sx
    ddy = pd[:, 1:2] - sy
    ddz = pd[:, 2:3] - sz
    d2 = (ddx * ddx + ddy * ddy) + ddz * ddz
    ci = jax.lax.broadcasted_iota(jnp.int32, (bd, ns), 1)
    cj = jax.lax.broadcasted_iota(jnp.int32, (bd, kk), 1)
    bigi = jnp.int32(2 ** 30)
    work0 = jnp.where(d2 <= r2, d2, POSINF)
    nbr0 = jnp.zeros((bd, kk), jnp.int32)
    dv0 = jnp.full((bd, kk), POSINF, jnp.float32)

    def body(j, st):
        work, nbr, dv = st
        v = jnp.min(work, axis=1, keepdims=True)
        a = jnp.min(jnp.where(work == v, ci, bigi), axis=1, keepdims=True)
        nbr = jnp.where(cj == j, a, nbr)
        dv = jnp.where(cj == j, v, dv)
        work = jnp.where(ci == a, POSINF, work)
        return work, nbr, dv

    _, nbr, dv = jax.lax.fori_loop(0, kk, body, (work0, nbr0, dv0))
    valid = dv <= r2
    nbr_ref[0] = jnp.where(valid, nbr, 0)
    vm_ref[0] = jnp.where(valid, 0.0, NEGINF)


def _nbr(ps, pd, r2, bd):
    mp = pd.shape[1]
    ns = ps.shape[2]
    grid = (NB, mp // bd)
    return pl.pallas_call(
        functools.partial(_nbr_kern, r2=r2, kk=KNB),
        grid=grid,
        in_specs=[pl.BlockSpec((1, 3, ns), lambda b, i: (b, 0, 0)),
                  pl.BlockSpec((1, bd, 3), lambda b, i: (b, i, 0))],
        out_specs=[pl.BlockSpec((1, bd, KNB), lambda b, i: (b, i, 0)),
                   pl.BlockSpec((1, bd, KNB), lambda b, i: (b, i, 0))],
        out_shape=[jax.ShapeDtypeStruct((NB, mp, KNB), jnp.int32),
                   jax.ShapeDtypeStruct((NB, mp, KNB), jnp.float32)],
    )(ps, pd)


def _sc_gather(table, idx, chunk, nch):
    vv, dd = table.shape
    bi = idx.shape[0]
    mesh = plsc.VectorSubcoreMesh(core_axis_name="c", subcore_axis_name="s")

    @functools.partial(
        pl.kernel, mesh=mesh,
        out_type=jax.ShapeDtypeStruct((bi, dd), jnp.float32),
        scratch_types=[pltpu.VMEM((chunk,), jnp.int32),
                       pltpu.VMEM((chunk, dd), jnp.float32),
                       pltpu.SemaphoreType.DMA],
    )
    def k(table_hbm, idx_hbm, out_hbm, idx_v, rows_v, sem):
        wid = jax.lax.axis_index("s") * 2 + jax.lax.axis_index("c")
        base = wid * (chunk * nch)
        for c in range(nch):
            off = base + c * chunk
            pltpu.sync_copy(idx_hbm.at[pl.ds(off, chunk)], idx_v)
            pltpu.async_copy(table_hbm.at[idx_v], rows_v, sem).wait()
            pltpu.sync_copy(rows_v, out_hbm.at[pl.ds(off, chunk)])

    return k(table, idx)


def _sa_mlp_kern(g_ref, pd_ref, vm_ref, w1_ref, w1r_ref, b1_ref, w2_ref,
                 b2_ref, w3_ref, b3_ref, out_ref, *, kk):
    bd = pd_ref.shape[1]
    g = g_ref[0]
    corr = jnp.dot(pd_ref[0], w1r_ref[...],
                   preferred_element_type=jnp.float32)
    h = jnp.dot(g, w1_ref[...], preferred_element_type=jnp.float32)
    h1 = h.reshape(bd, kk, -1) - corr[:, None, :] + b1_ref[...][None]
    h1 = jnp.maximum(h1, 0.0).reshape(bd * kk, -1)
    h2 = jnp.maximum(
        jnp.dot(h1, w2_ref[...], preferred_element_type=jnp.float32)
        + b2_ref[...], 0.0)
    h3 = jnp.maximum(
        jnp.dot(h2, w3_ref[...], preferred_element_type=jnp.float32)
        + b3_ref[...], 0.0)
    h3 = h3 + vm_ref[0]
    out_ref[0] = jnp.max(h3.reshape(bd, kk, -1), axis=1)


def _sa_mlp(g, pdp, vmf, prm, cin, dp, bd):
    (w1, b1), (w2, b2), (w3, b3) = prm
    h1 = w1.shape[1]
    h3 = w3.shape[1]
    mp = pdp.shape[1]
    w1p = jnp.zeros((dp, h1), jnp.float32).at[: cin + 3].set(w1)
    w1r = w1[cin: cin + 3]
    grid = (NB, mp // bd)
    return pl.pallas_call(
        functools.partial(_sa_mlp_kern, kk=KNB),
        grid=grid,
        in_specs=[
            pl.BlockSpec((1, bd * KNB, dp), lambda b, i: (b, i, 0)),
            pl.BlockSpec((1, bd, 3), lambda b, i: (b, i, 0)),
            pl.BlockSpec((1, bd * KNB, 1), lambda b, i: (b, i, 0)),
            pl.BlockSpec(w1p.shape, lambda b, i: (0, 0)),
            pl.BlockSpec(w1r.shape, lambda b, i: (0, 0)),
            pl.BlockSpec((1, h1), lambda b, i: (0, 0)),
            pl.BlockSpec(w2.shape, lambda b, i: (0, 0)),
            pl.BlockSpec((1, w2.shape[1]), lambda b, i: (0, 0)),
            pl.BlockSpec(w3.shape, lambda b, i: (0, 0)),
            pl.BlockSpec((1, h3), lambda b, i: (0, 0)),
        ],
        out_specs=pl.BlockSpec((1, bd, h3), lambda b, i: (b, i, 0)),
        out_shape=jax.ShapeDtypeStruct((NB, mp, h3), jnp.float32),
    )(g, pdp, vmf, w1p, w1r, b1[None], w2, b2[None], w3, b3[None])


def _knn3_kern(ps_ref, pd_ref, nbr_ref, wn_ref):
    ns = ps_ref.shape[2]
    md = pd_ref.shape[1]
    pd = pd_ref[0]
    ddx = pd[:, 0:1] - ps_ref[0, 0:1, :]
    ddy = pd[:, 1:2] - ps_ref[0, 1:2, :]
    ddz = pd[:, 2:3] - ps_ref[0, 2:3, :]
    work = (ddx * ddx + ddy * ddy) + ddz * ddz
    ci = jax.lax.broadcasted_iota(jnp.int32, (md, ns), 1)
    bigi = jnp.int32(2 ** 30)
    avals = []
    wvals = []
    for _ in range(3):
        v = jnp.min(work, axis=1, keepdims=True)
        a = jnp.min(jnp.where(work == v, ci, bigi), axis=1, keepdims=True)
        work = jnp.where(ci == a, POSINF, work)
        avals.append(a)
        wvals.append(1.0 / jnp.maximum(v, 1e-16))
    tot = (wvals[0] + wvals[1]) + wvals[2]
    for s in range(3):
        nbr_ref[0, :, s: s + 1] = avals[s]
        wn_ref[0, :, s: s + 1] = wvals[s] / tot


def _knn3(ps, pdp):
    ns = ps.shape[2]
    md = pdp.shape[1]
    return pl.pallas_call(
        _knn3_kern,
        grid=(NB,),
        in_specs=[pl.BlockSpec((1, 3, ns), lambda b: (b, 0, 0)),
                  pl.BlockSpec((1, md, 3), lambda b: (b, 0, 0))],
        out_specs=[pl.BlockSpec((1, md, 3), lambda b: (b, 0, 0)),
                   pl.BlockSpec((1, md, 3), lambda b: (b, 0, 0))],
        out_shape=[jax.ShapeDtypeStruct((NB, md, 3), jnp.int32),
                   jax.ShapeDtypeStruct((NB, md, 3), jnp.float32)],
    )(ps, pdp)


def _mlp_chain(h, wb_refs, relu_flags):
    nl = len(relu_flags)
    for i in range(nl):
        w = wb_refs[2 * i][...]
        b = wb_refs[2 * i + 1][...]
        h = jnp.dot(h, w, preferred_element_type=jnp.float32) + b
        if relu_flags[i]:
            h = jnp.maximum(h, 0.0)
    return h


def _mlp_rows_kern(x_ref, *args, relu_flags):
    out_ref = args[-1]
    out_ref[0] = _mlp_chain(x_ref[0], args[:-1], relu_flags)


def _mlp_rows(x, prm, bd, relu_flags=None):
    nl = len(prm)
    if relu_flags is None:
        relu_flags = [True] * nl
    mp = x.shape[1]
    cin = x.shape[2]
    hout = prm[-1][0].shape[1]
    wb = []
    specs = [pl.BlockSpec((1, bd, cin), lambda b, i: (b, i, 0))]
    for (w, b) in prm:
        wb += [w, b[None]]
        specs.append(pl.BlockSpec(w.shape, lambda b, i: (0, 0)))
        specs.append(pl.BlockSpec((1, w.shape[1]), lambda b, i: (0, 0)))
    return pl.pallas_call(
        functools.partial(_mlp_rows_kern, relu_flags=tuple(relu_flags)),
        grid=(NB, mp // bd),
        in_specs=specs,
        out_specs=pl.BlockSpec((1, bd, hout), lambda b, i: (b, i, 0)),
        out_shape=jax.ShapeDtypeStruct((NB, mp, hout), jnp.float32),
    )(x, *wb)


def _mlp_max_kern(x_ref, *args, relu_flags, nvalid):
    out_ref = args[-1]
    h = _mlp_chain(x_ref[0], args[:-1], relu_flags)
    ri = jax.lax.broadcasted_iota(jnp.int32, h.shape, 0)
    h = jnp.where(ri < nvalid, h, NEGINF)
    out_ref[0] = jnp.max(h, axis=0, keepdims=True)


def _mlp_max(x, prm, nvalid):
    nl = len(prm)
    mp = x.shape[1]
    cin = x.shape[2]
    hout = prm[-1][0].shape[1]
    wb = []
    specs = [pl.BlockSpec((1, mp, cin), lambda b: (b, 0, 0))]
    for (w, b) in prm:
        wb += [w, b[None]]
        specs.append(pl.BlockSpec(w.shape, lambda b: (0, 0)))
        specs.append(pl.BlockSpec((1, w.shape[1]), lambda b: (0, 0)))
    return pl.pallas_call(
        functools.partial(_mlp_max_kern, relu_flags=(True,) * nl,
                          nvalid=nvalid),
        grid=(NB,),
        in_specs=specs,
        out_specs=pl.BlockSpec((1, 1, hout), lambda b: (b, 0, 0)),
        out_shape=jax.ShapeDtypeStruct((NB, 1, hout), jnp.float32),
    )(x, *wb)


def _interp_mlp_kern(xg_ref, wn_ref, skip_ref, *args, relu_flags):
    out_ref = args[-1]
    bd = wn_ref.shape[1]
    cc = xg_ref.shape[2]
    xg = xg_ref[0].reshape(bd, 3, cc)
    wn = wn_ref[0]
    y = (xg[:, 0, :] * wn[:, 0:1] + xg[:, 1, :] * wn[:, 1:2]) \
        + xg[:, 2, :] * wn[:, 2:3]
    h = jnp.concatenate([y, skip_ref[0]], axis=1)
    out_ref[0] = _mlp_chain(h, args[:-1], relu_flags)


def _interp_mlp(xg, wn, skip, prm, bd, relu_flags=None):
    nl = len(prm)
    if relu_flags is None:
        relu_flags = [True] * nl
    mp = wn.shape[1]
    cc = xg.shape[2]
    cs = skip.shape[2]
    hout = prm[-1][0].shape[1]
    wb = []
    specs = [pl.BlockSpec((1, bd * 3, cc), lambda b, i: (b, i, 0)),
             pl.BlockSpec((1, bd, 3), lambda b, i: (b, i, 0)),
             pl.BlockSpec((1, bd, cs), lambda b, i: (b, i, 0))]
    for (w, b) in prm:
        wb += [w, b[None]]
        specs.append(pl.BlockSpec(w.shape, lambda b, i: (0, 0)))
        specs.append(pl.BlockSpec((1, w.shape[1]), lambda b, i: (0, 0)))
    return pl.pallas_call(
        functools.partial(_interp_mlp_kern, relu_flags=tuple(relu_flags)),
        grid=(NB, mp // bd),
        in_specs=specs,
        out_specs=pl.BlockSpec((1, bd, hout), lambda b, i: (b, i, 0)),
        out_shape=jax.ShapeDtypeStruct((NB, mp, hout), jnp.float32),
    )(xg, wn, skip, *wb)


def _pad_rows(a, mp, val):
    b, m, c = a.shape
    if m == mp:
        return a
    return jnp.concatenate(
        [a, jnp.full((b, mp - m, c), val, a.dtype)], axis=1)


def kernel(x, pos, batch, params):
    r2_1 = 0.2 * 0.2
    r2_2 = 0.4 * 0.4
    boff = jnp.arange(NB, dtype=jnp.int32)[:, None, None]

    x0 = x.reshape(NB, N0, IN_CH)
    p0 = pos.reshape(NB, N0, 3)
    p0t = jnp.transpose(p0, (0, 2, 1))

    idx1 = _fps(p0t.reshape(NB, 3, 8, N0 // 8), N0, M1, M1P)
    p1 = jnp.take_along_axis(p0, idx1[:, :, None], axis=1)
    p1p = _pad_rows(p1, M1P, PADV)
    nbr1, vm1 = _nbr(p0t, p1p, r2_1, bd=256)
    t1 = jnp.concatenate(
        [x0, p0, jnp.zeros((NB, N0, 119), jnp.float32)], axis=2)
    g1 = _sc_gather(t1.reshape(NB * N0, 128),
                    (nbr1 + boff * N0).reshape(-1), 512, 16)
    x1 = _sa_mlp(g1.reshape(NB, M1P * KNB, 128), p1p,
                 vm1.reshape(NB, M1P * KNB, 1),
                 params['sa1'], cin=IN_CH, dp=128, bd=128)

    p1pt = jnp.transpose(p1p, (0, 2, 1))
    idx2 = _fps(p1pt.reshape(NB, 3, 8, M1P // 8), M1, M2, M2P)
    p2 = jnp.take_along_axis(p1, idx2[:, :, None], axis=1)
    p2p = _pad_rows(p2, M2P, PADV)
    nbr2, vm2 = _nbr(p1pt, p2p, r2_2, bd=256)
    t2 = jnp.concatenate(
        [x1, p1p, jnp.zeros((NB, M1P, 125), jnp.float32)], axis=2)
    g2 = _sc_gather(t2.reshape(NB * M1P, 256),
                    (nbr2 + boff * M1P).reshape(-1), 256, 8)
    x2 = _sa_mlp(g2.reshape(NB, M2P * KNB, 256), p2p,
                 vm2.reshape(NB, M2P * KNB, 1),
                 params['sa2'], cin=128, dp=256, bd=128)

    x3 = _mlp_max(jnp.concatenate([x2, p2p], axis=2), params['sa3'],
                  nvalid=M2)
    y2 = _mlp_rows(
        jnp.concatenate([jnp.broadcast_to(x3, (NB, M2P, x3.shape[2])), x2],
                        axis=2),
        params['fp3'], bd=M2P)

    p2pt = jnp.transpose(p2p, (0, 2, 1))
    nbr3a, wn3a = _knn3(p2pt, p1p)
    xka = _sc_gather(y2.reshape(NB * M2P, 256),
                     (nbr3a + boff * M2P).reshape(-1), 128, 3)
    y1 = _interp_mlp(xka.reshape(NB, M1P * 3, 256), wn3a, x1,
                     params['fp2'], bd=256)

    nbr3b, wn3b = _knn3(p1pt, p0)
    xkb = _sc_gather(y1.reshape(NB * M1P, 128),
                     (nbr3b + boff * M1P).reshape(-1), 512, 3)
    out = _interp_mlp(xkb.reshape(NB, N0 * 3, 128), wn3b, x0,
                      params['fp1'] + params['head'], bd=512,
                      relu_flags=[True] * 5 + [False])
    return out.reshape(NB * N0, 13)
